# baseline (speedup 1.0000x reference)
import functools

import jax
import jax.numpy as jnp
from jax import lax
from jax.experimental import pallas as pl
from jax.experimental.pallas import tpu as pltpu
from jax.experimental.pallas import tpu_sc as plsc

N = 2048
C = 768
F = 64
E = 8
H = 1536
NB = 4
R = 8
ALPHA = 16.0
CAP = 1.25
LOSS_COEF = 0.01
SCALE = ALPHA / float(R)
K_SEL = 320
LR = NB * R

TILE = 128
NT = 40
P = NT * TILE

SC_CORES = 2
SC_SUBCORES = 16
NW = SC_CORES * SC_SUBCORES
RW = N // NW
CT = 32


def _router_body(lg_ref, pos1_ref, pos2_ref, wp1_ref, wp2_ref, te_ref,
                 loss_ref, rk_ref):
    lg = lg_ref[...]
    u = lax.bitcast_convert_type(lg, jnp.uint32)
    key = jnp.where((u >> 31) == jnp.uint32(1), ~u,
                    u ^ jnp.uint32(0x80000000))

    t = jnp.zeros((1, E), jnp.uint32)
    for b in range(31, -1, -1):
        cand = t | jnp.uint32(1 << b)
        cnt = jnp.sum((key >= cand).astype(jnp.float32), axis=0, keepdims=True)
        t = jnp.where(cnt >= K_SEL, cand, t)

    gt = key > t
    n_gt = jnp.sum(gt.astype(jnp.float32), axis=0, keepdims=True)
    eq = key == t

    BLK = 256
    rio = lax.broadcasted_iota(jnp.int32, (BLK, BLK), 0)
    cio = lax.broadcasted_iota(jnp.int32, (BLK, BLK), 1)
    LT = (rio > cio).astype(jnp.float32)

    def colcumsum_excl(m):
        carry = jnp.zeros((1, E), jnp.float32)
        for i in range(N // BLK):
            blk = m[i * BLK:(i + 1) * BLK, :]
            rk_ref[i * BLK:(i + 1) * BLK, :] = (
                jnp.dot(LT, blk, preferred_element_type=jnp.float32) + carry)
            carry = carry + jnp.sum(blk, axis=0, keepdims=True)
        return rk_ref[...], carry

    eq_rank, _ = colcumsum_excl(eq.astype(jnp.float32))
    need = jnp.float32(K_SEL) - n_gt
    dispatch = gt | (eq & (eq_rank < need))

    e_iota = lax.broadcasted_iota(jnp.int32, (N, E), 1).astype(jnp.float32)
    covered = jnp.max(dispatch.astype(jnp.float32), axis=1, keepdims=True) > 0.0
    rowmax = jnp.max(lg, axis=1, keepdims=True)
    best = jnp.min(jnp.where(lg == rowmax, e_iota, 1e9), axis=1, keepdims=True)
    dispatch = dispatch | ((~covered) & (e_iota == best))

    neg = jnp.float32(-jnp.inf)
    masked = jnp.where(dispatch, lg, neg)
    m1 = jnp.max(masked, axis=1, keepdims=True)
    i1 = jnp.min(jnp.where(masked == m1, e_iota, 1e9), axis=1, keepdims=True)
    v2 = jnp.where(e_iota == i1, neg, masked)
    m2 = jnp.max(v2, axis=1, keepdims=True)
    i2 = jnp.min(jnp.where((v2 == m2) & (e_iota != i1), e_iota, 1e9),
                 axis=1, keepdims=True)

    one1 = e_iota == i1
    one2 = e_iota == i2
    l1 = jnp.sum(jnp.where(one1, lg, 0.0), axis=1, keepdims=True)
    l2 = jnp.sum(jnp.where(one2, lg, 0.0), axis=1, keepdims=True)
    mx = jnp.maximum(l1, l2)
    ex1 = jnp.exp(l1 - mx)
    ex2 = jnp.exp(l2 - mx)
    s = ex1 + ex2
    w1 = ex1 / s
    w2 = ex2 / s

    member = (one1 | one2).astype(jnp.float32)
    rank, counts = colcumsum_excl(member)
    seg = jnp.floor((counts + jnp.float32(TILE - 1)) / TILE) * TILE

    rio8 = lax.broadcasted_iota(jnp.int32, (E, E), 0)
    cio8 = lax.broadcasted_iota(jnp.int32, (E, E), 1)
    slt8 = (rio8 < cio8).astype(jnp.float32)
    offs = jnp.dot(seg, slt8, preferred_element_type=jnp.float32)

    o1 = jnp.sum(jnp.where(one1, offs, 0.0), axis=1, keepdims=True)
    o2 = jnp.sum(jnp.where(one2, offs, 0.0), axis=1, keepdims=True)
    r1 = jnp.sum(jnp.where(one1, rank, 0.0), axis=1, keepdims=True)
    r2 = jnp.sum(jnp.where(one2, rank, 0.0), axis=1, keepdims=True)
    pos1_ref[...] = (o1 + r1).astype(jnp.int32)
    pos2_ref[...] = (o2 + r2).astype(jnp.int32)
    wp1_ref[...] = jnp.broadcast_to(w1, (N, E))
    wp2_ref[...] = jnp.broadcast_to(w2, (N, E))

    ends = offs + seg
    ts = lax.broadcasted_iota(jnp.int32, (64, E), 0).astype(jnp.float32) * TILE
    te = jnp.sum((ts >= ends).astype(jnp.float32), axis=1, keepdims=True)
    te_ref[...] = jnp.minimum(te, jnp.float32(E - 1)).astype(jnp.int32)

    imp = jnp.sum(jnp.where(one1, w1, 0.0) + jnp.where(one2, w2, 0.0),
                  axis=0, keepdims=True)
    load = counts

    def cv(v):
        m = jnp.mean(v, axis=1, keepdims=True)
        var = jnp.mean((v - m) ** 2, axis=1, keepdims=True)
        return var / (m * m + 1e-10)

    loss_ref[...] = (cv(imp) + cv(load)) * LOSS_COEF


def _router(logits):
    return pl.pallas_call(
        _router_body,
        out_shape=[
            jax.ShapeDtypeStruct((N, 1), jnp.int32),
            jax.ShapeDtypeStruct((N, 1), jnp.int32),
            jax.ShapeDtypeStruct((N, E), jnp.float32),
            jax.ShapeDtypeStruct((N, E), jnp.float32),
            jax.ShapeDtypeStruct((64, 1), jnp.int32),
            jax.ShapeDtypeStruct((1, 1), jnp.float32),
        ],
        scratch_shapes=[pltpu.VMEM((N, E), jnp.float32)],
    )(logits)


@functools.lru_cache(maxsize=1)
def _sc_mesh():
    return plsc.VectorSubcoreMesh(
        core_axis_name="c", subcore_axis_name="s",
        num_cores=SC_CORES, num_subcores=SC_SUBCORES)


@functools.lru_cache(maxsize=1)
def _sc_scatter():
    @functools.partial(
        pl.kernel, mesh=_sc_mesh(),
        out_type=[
            jax.ShapeDtypeStruct((P, C), jnp.float32),
            jax.ShapeDtypeStruct((P, 128), jnp.float32),
        ],
        scratch_types=[
            pltpu.VMEM((RW,), jnp.int32),
            pltpu.VMEM((RW, C), jnp.float32),
            pltpu.VMEM((RW, 128), jnp.float32),
            pltpu.SemaphoreType.DMA,
        ],
    )
    def body(x_hbm, cat1_hbm, cat2_hbm, pos1_hbm, pos2_hbm,
             ys_hbm, meta_hbm, idx_v, rows_v, mt_v, sem):
        wid = lax.axis_index("s") * SC_CORES + lax.axis_index("c")
        base = wid * RW
        pltpu.sync_copy(x_hbm.at[pl.ds(base, RW)], rows_v)
        pltpu.sync_copy(pos1_hbm.at[pl.ds(base, RW)], idx_v)
        pltpu.sync_copy(cat1_hbm.at[pl.ds(base, RW)], mt_v)
        pltpu.async_copy(rows_v, ys_hbm.at[idx_v], sem).wait()
        pltpu.async_copy(mt_v, meta_hbm.at[idx_v], sem).wait()
        pltpu.sync_copy(pos2_hbm.at[pl.ds(base, RW)], idx_v)
        pltpu.sync_copy(cat2_hbm.at[pl.ds(base, RW)], mt_v)
        pltpu.async_copy(rows_v, ys_hbm.at[idx_v], sem).wait()
        pltpu.async_copy(mt_v, meta_hbm.at[idx_v], sem).wait()

    return body


def _scatter_call(x32, cat1, cat2, pos1f, pos2f):
    return _sc_scatter()(x32, cat1, cat2, pos1f, pos2f)


def _ffn_body(te_ref, ys_ref, meta_ref, W1_ref, b1_ref, A1_ref,
              B1_ref, W2_ref, b2_ref, A2_ref, B2_ref, os_ref):
    ys = ys_ref[...]
    ysb = ys.astype(jnp.bfloat16)
    meta = meta_ref[...]
    bwr = meta[:, :LR]
    xa = jnp.dot(ysb, A1_ref[0], preferred_element_type=jnp.float32) * bwr
    h = (jnp.dot(ysb, W1_ref[0], preferred_element_type=jnp.float32)
         + b1_ref[0]
         + jnp.dot(xa.astype(jnp.bfloat16), B1_ref[0],
                   preferred_element_type=jnp.float32))
    h = jax.nn.gelu(h)
    hb = h.astype(jnp.bfloat16)
    ha = jnp.dot(hb, A2_ref[0], preferred_element_type=jnp.float32) * bwr
    out = (jnp.dot(hb, W2_ref[0], preferred_element_type=jnp.float32)
           + b2_ref[0]
           + jnp.dot(ha.astype(jnp.bfloat16), B2_ref[0],
                     preferred_element_type=jnp.float32))
    os_ref[...] = out * meta[:, LR:LR + 1]


def _ffn(tef, ys, meta, W1, b1r, A1c, B1c, W2, b2r, A2c, B2c):
    grid_spec = pltpu.PrefetchScalarGridSpec(
        num_scalar_prefetch=1,
        grid=(NT,),
        in_specs=[
            pl.BlockSpec((TILE, C), lambda i, te: (i, 0)),
            pl.BlockSpec((TILE, 128), lambda i, te: (i, 0)),
            pl.BlockSpec((1, C, H), lambda i, te: (te[i], 0, 0)),
            pl.BlockSpec((1, 1, H), lambda i, te: (te[i], 0, 0)),
            pl.BlockSpec((1, C, LR), lambda i, te: (te[i], 0, 0)),
            pl.BlockSpec((1, LR, H), lambda i, te: (te[i], 0, 0)),
            pl.BlockSpec((1, H, C), lambda i, te: (te[i], 0, 0)),
            pl.BlockSpec((1, 1, C), lambda i, te: (te[i], 0, 0)),
            pl.BlockSpec((1, H, LR), lambda i, te: (te[i], 0, 0)),
            pl.BlockSpec((1, LR, C), lambda i, te: (te[i], 0, 0)),
        ],
        out_specs=pl.BlockSpec((TILE, C), lambda i, te: (i, 0)),
    )
    return pl.pallas_call(
        _ffn_body,
        grid_spec=grid_spec,
        out_shape=jax.ShapeDtypeStruct((P, C), jnp.float32),
    )(tef, ys, meta, W1, b1r, A1c, B1c, W2, b2r, A2c, B2c)


@functools.lru_cache(maxsize=1)
def _sc_combine():
    @functools.partial(
        pl.kernel, mesh=_sc_mesh(),
        out_type=jax.ShapeDtypeStruct((N, C), jnp.float32),
        scratch_types=[
            pltpu.VMEM((CT,), jnp.int32),
            pltpu.VMEM((CT,), jnp.int32),
            pltpu.VMEM((CT, C), jnp.float32),
            pltpu.VMEM((CT, C), jnp.float32),
            pltpu.VMEM((CT, C), jnp.float32),
            pltpu.SemaphoreType.DMA,
            pltpu.SemaphoreType.DMA,
        ],
    )
    def body(os_hbm, pos1_hbm, pos2_hbm, fin_hbm,
             i1_v, i2_v, b1_v, b2_v, o_v, s1, s2):
        wid = lax.axis_index("s") * SC_CORES + lax.axis_index("c")
        for c in range(RW // CT):
            base = wid * RW + c * CT
            pltpu.sync_copy(pos1_hbm.at[pl.ds(base, CT)], i1_v)
            pltpu.sync_copy(pos2_hbm.at[pl.ds(base, CT)], i2_v)
            cp1 = pltpu.async_copy(os_hbm.at[i1_v], b1_v, s1)
            cp2 = pltpu.async_copy(os_hbm.at[i2_v], b2_v, s2)
            cp1.wait()
            cp2.wait()

            def tok(ti, _):
                def vv(j, __):
                    o_v[ti, pl.ds(j * 16, 16)] = (
                        b1_v[ti, pl.ds(j * 16, 16)]
                        + b2_v[ti, pl.ds(j * 16, 16)])
                    return 0
                return lax.fori_loop(0, C // 16, vv, 0)

            lax.fori_loop(0, CT, tok, 0)
            pltpu.sync_copy(o_v, fin_hbm.at[pl.ds(base, CT)])

    return body


def _combine_call(os, pos1f, pos2f):
    return _sc_combine()(os, pos1f, pos2f)


def kernel(x, band_weights, x_prev_tokens, W_ext, ln_g, ln_b, W_gate, b_gate,
           W1, b1, A1, B1, W2, b2, A2, B2):
    x32 = x.astype(jnp.float32)
    Z = x32 @ W_ext.T
    delta = jax.lax.stop_gradient(x32 - x_prev_tokens.astype(jnp.float32))
    ad = jnp.abs(delta)
    mu = jnp.log1p(jnp.mean(ad, axis=1, keepdims=True))
    sd = jnp.log1p(jnp.std(ad, axis=1, keepdims=True, ddof=1))
    enh = jnp.concatenate([x32, Z, mu, sd], axis=1)
    mean = jnp.mean(enh, axis=-1, keepdims=True)
    var = jnp.var(enh, axis=-1, keepdims=True)
    enh = (enh - mean) / jnp.sqrt(var + 1e-5) * ln_g + ln_b
    logits = enh @ W_gate.T + b_gate

    pos1, pos2, wp1, wp2, te, lossv = _router(logits)
    pos1f = pos1.reshape(N)
    pos2f = pos2.reshape(N)
    tef = te.reshape(64)

    A1c = jnp.transpose(A1, (0, 2, 1, 3)).reshape(E, C, LR).astype(jnp.bfloat16)
    B1c = (B1.reshape(E, LR, H) * SCALE).astype(jnp.bfloat16)
    A2c = jnp.transpose(A2, (0, 2, 1, 3)).reshape(E, H, LR).astype(jnp.bfloat16)
    B2c = (B2.reshape(E, LR, C) * SCALE).astype(jnp.bfloat16)
    bwr = jnp.repeat(band_weights, R, axis=1)
    pad = jnp.zeros((N, 128 - LR - E), jnp.float32)
    cat1 = jnp.concatenate([bwr, wp1, pad], axis=1)
    cat2 = jnp.concatenate([bwr, wp2, pad], axis=1)

    ys, meta = _scatter_call(x32, cat1, cat2, pos1f, pos2f)
    os = _ffn(tef, ys, meta, W1.astype(jnp.bfloat16), b1.reshape(E, 1, H),
              A1c, B1c, W2.astype(jnp.bfloat16), b2.reshape(E, 1, C),
              A2c, B2c)
    final = _combine_call(os, pos1f, pos2f)
    return final, lossv.reshape(())


# speedup vs baseline: 1.2133x; 1.2133x over previous
import functools

import jax
import jax.numpy as jnp
from jax import lax
from jax.experimental import pallas as pl
from jax.experimental.pallas import tpu as pltpu
from jax.experimental.pallas import tpu_sc as plsc

N = 2048
C = 768
F = 64
E = 8
H = 1536
NB = 4
R = 8
ALPHA = 16.0
CAP = 1.25
LOSS_COEF = 0.01
SCALE = ALPHA / float(R)
K_SEL = 320
LR = NB * R

TILE = 256
NT = 24
P = NT * TILE

SC_CORES = 2
SC_SUBCORES = 16
NW = SC_CORES * SC_SUBCORES
RW = N // NW
CT = 32


def _router_body(lg_ref, bwr_ref, pos1_ref, pos2_ref, cat1_ref, cat2_ref,
                 te_ref, loss_ref, rk_ref):
    lg = lg_ref[...]
    u = lax.bitcast_convert_type(lg, jnp.uint32)
    key = jnp.where((u >> 31) == jnp.uint32(1), ~u,
                    u ^ jnp.uint32(0x80000000))

    t = jnp.zeros((1, E), jnp.uint32)
    for b in range(31, -1, -1):
        cand = t | jnp.uint32(1 << b)
        cnt = jnp.sum((key >= cand).astype(jnp.float32), axis=0, keepdims=True)
        t = jnp.where(cnt >= K_SEL, cand, t)

    gt = key > t
    n_gt = jnp.sum(gt.astype(jnp.float32), axis=0, keepdims=True)
    eq = key == t

    BLK = 256
    rio = lax.broadcasted_iota(jnp.int32, (BLK, BLK), 0)
    cio = lax.broadcasted_iota(jnp.int32, (BLK, BLK), 1)
    LT = (rio > cio).astype(jnp.float32)

    def colcumsum_excl(m):
        carry = jnp.zeros((1, E), jnp.float32)
        for i in range(N // BLK):
            blk = m[i * BLK:(i + 1) * BLK, :]
            rk_ref[i * BLK:(i + 1) * BLK, :] = (
                jnp.dot(LT, blk, preferred_element_type=jnp.float32) + carry)
            carry = carry + jnp.sum(blk, axis=0, keepdims=True)
        return rk_ref[...], carry

    eq_rank, _ = colcumsum_excl(eq.astype(jnp.float32))
    need = jnp.float32(K_SEL) - n_gt
    dispatch = gt | (eq & (eq_rank < need))

    e_iota = lax.broadcasted_iota(jnp.int32, (N, E), 1).astype(jnp.float32)
    covered = jnp.max(dispatch.astype(jnp.float32), axis=1, keepdims=True) > 0.0
    rowmax = jnp.max(lg, axis=1, keepdims=True)
    best = jnp.min(jnp.where(lg == rowmax, e_iota, 1e9), axis=1, keepdims=True)
    dispatch = dispatch | ((~covered) & (e_iota == best))

    neg = jnp.float32(-jnp.inf)
    masked = jnp.where(dispatch, lg, neg)
    m1 = jnp.max(masked, axis=1, keepdims=True)
    i1 = jnp.min(jnp.where(masked == m1, e_iota, 1e9), axis=1, keepdims=True)
    v2 = jnp.where(e_iota == i1, neg, masked)
    m2 = jnp.max(v2, axis=1, keepdims=True)
    i2 = jnp.min(jnp.where((v2 == m2) & (e_iota != i1), e_iota, 1e9),
                 axis=1, keepdims=True)

    one1 = e_iota == i1
    one2 = e_iota == i2
    l1 = jnp.sum(jnp.where(one1, lg, 0.0), axis=1, keepdims=True)
    l2 = jnp.sum(jnp.where(one2, lg, 0.0), axis=1, keepdims=True)
    mx = jnp.maximum(l1, l2)
    ex1 = jnp.exp(l1 - mx)
    ex2 = jnp.exp(l2 - mx)
    s = ex1 + ex2
    w1 = ex1 / s
    w2 = ex2 / s

    member = (one1 | one2).astype(jnp.float32)
    rank, counts = colcumsum_excl(member)
    seg = jnp.floor((counts + jnp.float32(TILE - 1)) / TILE) * TILE

    rio8 = lax.broadcasted_iota(jnp.int32, (E, E), 0)
    cio8 = lax.broadcasted_iota(jnp.int32, (E, E), 1)
    slt8 = (rio8 < cio8).astype(jnp.float32)
    offs = jnp.dot(seg, slt8, preferred_element_type=jnp.float32)

    o1 = jnp.sum(jnp.where(one1, offs, 0.0), axis=1, keepdims=True)
    o2 = jnp.sum(jnp.where(one2, offs, 0.0), axis=1, keepdims=True)
    r1 = jnp.sum(jnp.where(one1, rank, 0.0), axis=1, keepdims=True)
    r2 = jnp.sum(jnp.where(one2, rank, 0.0), axis=1, keepdims=True)
    pos1_ref[...] = (o1 + r1).astype(jnp.int32)
    pos2_ref[...] = (o2 + r2).astype(jnp.int32)
    bwr = bwr_ref[...]
    zpad = jnp.zeros((N, 128 - LR - E), jnp.float32)
    cat1_ref[...] = jnp.concatenate(
        [bwr, jnp.broadcast_to(w1, (N, E)), zpad], axis=1)
    cat2_ref[...] = jnp.concatenate(
        [bwr, jnp.broadcast_to(w2, (N, E)), zpad], axis=1)

    ends = offs + seg
    ts = lax.broadcasted_iota(jnp.int32, (64, E), 0).astype(jnp.float32) * TILE
    te = jnp.sum((ts >= ends).astype(jnp.float32), axis=1, keepdims=True)
    te_ref[...] = jnp.minimum(te, jnp.float32(E - 1)).astype(jnp.int32)

    imp = jnp.sum(jnp.where(one1, w1, 0.0) + jnp.where(one2, w2, 0.0),
                  axis=0, keepdims=True)
    load = counts

    def cv(v):
        m = jnp.mean(v, axis=1, keepdims=True)
        var = jnp.mean((v - m) ** 2, axis=1, keepdims=True)
        return var / (m * m + 1e-10)

    loss_ref[...] = (cv(imp) + cv(load)) * LOSS_COEF


def _router(logits, bwr):
    return pl.pallas_call(
        _router_body,
        out_shape=[
            jax.ShapeDtypeStruct((N, 1), jnp.int32),
            jax.ShapeDtypeStruct((N, 1), jnp.int32),
            jax.ShapeDtypeStruct((N, 128), jnp.float32),
            jax.ShapeDtypeStruct((N, 128), jnp.float32),
            jax.ShapeDtypeStruct((64, 1), jnp.int32),
            jax.ShapeDtypeStruct((1, 1), jnp.float32),
        ],
        scratch_shapes=[pltpu.VMEM((N, E), jnp.float32)],
    )(logits, bwr)


@functools.lru_cache(maxsize=1)
def _sc_mesh():
    return plsc.VectorSubcoreMesh(
        core_axis_name="c", subcore_axis_name="s",
        num_cores=SC_CORES, num_subcores=SC_SUBCORES)


@functools.lru_cache(maxsize=1)
def _sc_scatter():
    @functools.partial(
        pl.kernel, mesh=_sc_mesh(),
        out_type=[
            jax.ShapeDtypeStruct((P, C), jnp.float32),
            jax.ShapeDtypeStruct((P, 128), jnp.float32),
        ],
        scratch_types=[
            pltpu.VMEM((RW,), jnp.int32),
            pltpu.VMEM((RW, C), jnp.float32),
            pltpu.VMEM((RW, 128), jnp.float32),
            pltpu.SemaphoreType.DMA,
        ],
    )
    def body(x_hbm, cat1_hbm, cat2_hbm, pos1_hbm, pos2_hbm,
             ys_hbm, meta_hbm, idx_v, rows_v, mt_v, sem):
        wid = lax.axis_index("s") * SC_CORES + lax.axis_index("c")
        base = wid * RW
        pltpu.sync_copy(x_hbm.at[pl.ds(base, RW)], rows_v)
        pltpu.sync_copy(pos1_hbm.at[pl.ds(base, RW)], idx_v)
        pltpu.sync_copy(cat1_hbm.at[pl.ds(base, RW)], mt_v)
        pltpu.async_copy(rows_v, ys_hbm.at[idx_v], sem).wait()
        pltpu.async_copy(mt_v, meta_hbm.at[idx_v], sem).wait()
        pltpu.sync_copy(pos2_hbm.at[pl.ds(base, RW)], idx_v)
        pltpu.sync_copy(cat2_hbm.at[pl.ds(base, RW)], mt_v)
        pltpu.async_copy(rows_v, ys_hbm.at[idx_v], sem).wait()
        pltpu.async_copy(mt_v, meta_hbm.at[idx_v], sem).wait()

    return body


def _scatter_call(x32, cat1, cat2, pos1f, pos2f):
    return _sc_scatter()(x32, cat1, cat2, pos1f, pos2f)


def _ffn_body(te_ref, ys_ref, meta_ref, W1_ref, b1_ref, A1_ref,
              B1_ref, W2_ref, b2_ref, A2_ref, B2_ref, os_ref):
    ys = ys_ref[...]
    meta = meta_ref[...]
    bwr = meta[:, :LR]
    xa = jnp.dot(ys, A1_ref[0], preferred_element_type=jnp.float32) * bwr
    h = (jnp.dot(ys, W1_ref[0], preferred_element_type=jnp.float32)
         + b1_ref[0]
         + jnp.dot(xa, B1_ref[0], preferred_element_type=jnp.float32))
    h = jax.nn.gelu(h)
    ha = jnp.dot(h, A2_ref[0], preferred_element_type=jnp.float32) * bwr
    out = (jnp.dot(h, W2_ref[0], preferred_element_type=jnp.float32)
           + b2_ref[0]
           + jnp.dot(ha, B2_ref[0], preferred_element_type=jnp.float32))
    os_ref[...] = out * meta[:, LR:LR + 1]


def _ffn(tef, ys, meta, W1, b1r, A1c, B1c, W2, b2r, A2c, B2c):
    grid_spec = pltpu.PrefetchScalarGridSpec(
        num_scalar_prefetch=1,
        grid=(NT,),
        in_specs=[
            pl.BlockSpec((TILE, C), lambda i, te: (i, 0)),
            pl.BlockSpec((TILE, 128), lambda i, te: (i, 0)),
            pl.BlockSpec((1, C, H), lambda i, te: (te[i], 0, 0)),
            pl.BlockSpec((1, 1, H), lambda i, te: (te[i], 0, 0)),
            pl.BlockSpec((1, C, LR), lambda i, te: (te[i], 0, 0)),
            pl.BlockSpec((1, LR, H), lambda i, te: (te[i], 0, 0)),
            pl.BlockSpec((1, H, C), lambda i, te: (te[i], 0, 0)),
            pl.BlockSpec((1, 1, C), lambda i, te: (te[i], 0, 0)),
            pl.BlockSpec((1, H, LR), lambda i, te: (te[i], 0, 0)),
            pl.BlockSpec((1, LR, C), lambda i, te: (te[i], 0, 0)),
        ],
        out_specs=pl.BlockSpec((TILE, C), lambda i, te: (i, 0)),
    )
    return pl.pallas_call(
        _ffn_body,
        grid_spec=grid_spec,
        out_shape=jax.ShapeDtypeStruct((P, C), jnp.float32),
    )(tef, ys, meta, W1, b1r, A1c, B1c, W2, b2r, A2c, B2c)


@functools.lru_cache(maxsize=1)
def _sc_combine():
    @functools.partial(
        pl.kernel, mesh=_sc_mesh(),
        out_type=jax.ShapeDtypeStruct((N, C), jnp.float32),
        scratch_types=[
            pltpu.VMEM((CT,), jnp.int32),
            pltpu.VMEM((CT,), jnp.int32),
            pltpu.VMEM((CT, C), jnp.float32),
            pltpu.VMEM((CT, C), jnp.float32),
            pltpu.VMEM((CT, C), jnp.float32),
            pltpu.SemaphoreType.DMA,
            pltpu.SemaphoreType.DMA,
        ],
    )
    def body(os_hbm, pos1_hbm, pos2_hbm, fin_hbm,
             i1_v, i2_v, b1_v, b2_v, o_v, s1, s2):
        wid = lax.axis_index("s") * SC_CORES + lax.axis_index("c")
        for c in range(RW // CT):
            base = wid * RW + c * CT
            pltpu.sync_copy(pos1_hbm.at[pl.ds(base, CT)], i1_v)
            pltpu.sync_copy(pos2_hbm.at[pl.ds(base, CT)], i2_v)
            cp1 = pltpu.async_copy(os_hbm.at[i1_v], b1_v, s1)
            cp2 = pltpu.async_copy(os_hbm.at[i2_v], b2_v, s2)
            cp1.wait()
            cp2.wait()

            def tok(ti, _):
                for j in range(C // 16):
                    o_v[ti, j * 16:(j + 1) * 16] = (
                        b1_v[ti, j * 16:(j + 1) * 16]
                        + b2_v[ti, j * 16:(j + 1) * 16])
                return 0

            lax.fori_loop(0, CT, tok, 0)
            pltpu.sync_copy(o_v, fin_hbm.at[pl.ds(base, CT)])

    return body


def _combine_call(os, pos1f, pos2f):
    return _sc_combine()(os, pos1f, pos2f)


def kernel(x, band_weights, x_prev_tokens, W_ext, ln_g, ln_b, W_gate, b_gate,
           W1, b1, A1, B1, W2, b2, A2, B2):
    x32 = x.astype(jnp.float32)
    Z = x32 @ W_ext.T
    delta = jax.lax.stop_gradient(x32 - x_prev_tokens.astype(jnp.float32))
    ad = jnp.abs(delta)
    mu = jnp.log1p(jnp.mean(ad, axis=1, keepdims=True))
    sd = jnp.log1p(jnp.std(ad, axis=1, keepdims=True, ddof=1))
    enh = jnp.concatenate([x32, Z, mu, sd], axis=1)
    mean = jnp.mean(enh, axis=-1, keepdims=True)
    var = jnp.var(enh, axis=-1, keepdims=True)
    enh = (enh - mean) / jnp.sqrt(var + 1e-5) * ln_g + ln_b
    logits = enh @ W_gate.T + b_gate

    bwr = jnp.repeat(band_weights, R, axis=1)
    pos1, pos2, cat1, cat2, te, lossv = _router(logits, bwr)
    pos1f = pos1.reshape(N)
    pos2f = pos2.reshape(N)
    tef = te.reshape(64)

    A1c = jnp.transpose(A1, (0, 2, 1, 3)).reshape(E, C, LR)
    B1c = B1.reshape(E, LR, H) * SCALE
    A2c = jnp.transpose(A2, (0, 2, 1, 3)).reshape(E, H, LR)
    B2c = B2.reshape(E, LR, C) * SCALE
    ys, meta = _scatter_call(x32, cat1, cat2, pos1f, pos2f)
    os = _ffn(tef, ys, meta, W1, b1.reshape(E, 1, H), A1c, B1c,
              W2, b2.reshape(E, 1, C), A2c, B2c)
    final = _combine_call(os, pos1f, pos2f)
    return final, lossv.reshape(())


# speedup vs baseline: 1.2148x; 1.0013x over previous
import functools

import jax
import jax.numpy as jnp
from jax import lax
from jax.experimental import pallas as pl
from jax.experimental.pallas import tpu as pltpu
from jax.experimental.pallas import tpu_sc as plsc

N = 2048
C = 768
F = 64
E = 8
H = 1536
NB = 4
R = 8
ALPHA = 16.0
CAP = 1.25
LOSS_COEF = 0.01
SCALE = ALPHA / float(R)
K_SEL = 320
LR = NB * R

TILE = 256
NT = 24
P = NT * TILE

SC_CORES = 2
SC_SUBCORES = 16
NW = SC_CORES * SC_SUBCORES
RW = N // NW
CT = 32


def _router_body(lg_ref, bwr_ref, pos1_ref, pos2_ref, cat1_ref, cat2_ref,
                 te_ref, loss_ref, rk_ref):
    lg = lg_ref[...]
    u = lax.bitcast_convert_type(lg, jnp.uint32)
    key = jnp.where((u >> 31) == jnp.uint32(1), ~u,
                    u ^ jnp.uint32(0x80000000))

    t = jnp.zeros((1, E), jnp.uint32)
    for b in range(31, -1, -1):
        cand = t | jnp.uint32(1 << b)
        cnt = jnp.sum((key >= cand).astype(jnp.float32), axis=0, keepdims=True)
        t = jnp.where(cnt >= K_SEL, cand, t)

    gt = key > t
    n_gt = jnp.sum(gt.astype(jnp.float32), axis=0, keepdims=True)
    eq = key == t

    BLK = 256
    rio = lax.broadcasted_iota(jnp.int32, (BLK, BLK), 0)
    cio = lax.broadcasted_iota(jnp.int32, (BLK, BLK), 1)
    LT = (rio > cio).astype(jnp.float32)

    def colcumsum_excl(m):
        carry = jnp.zeros((1, E), jnp.float32)
        for i in range(N // BLK):
            blk = m[i * BLK:(i + 1) * BLK, :]
            rk_ref[i * BLK:(i + 1) * BLK, :] = (
                jnp.dot(LT, blk, preferred_element_type=jnp.float32) + carry)
            carry = carry + jnp.sum(blk, axis=0, keepdims=True)
        return rk_ref[...], carry

    eq_rank, _ = colcumsum_excl(eq.astype(jnp.float32))
    need = jnp.float32(K_SEL) - n_gt
    dispatch = gt | (eq & (eq_rank < need))

    e_iota = lax.broadcasted_iota(jnp.int32, (N, E), 1).astype(jnp.float32)
    covered = jnp.max(dispatch.astype(jnp.float32), axis=1, keepdims=True) > 0.0
    rowmax = jnp.max(lg, axis=1, keepdims=True)
    best = jnp.min(jnp.where(lg == rowmax, e_iota, 1e9), axis=1, keepdims=True)
    dispatch = dispatch | ((~covered) & (e_iota == best))

    neg = jnp.float32(-jnp.inf)
    masked = jnp.where(dispatch, lg, neg)
    m1 = jnp.max(masked, axis=1, keepdims=True)
    i1 = jnp.min(jnp.where(masked == m1, e_iota, 1e9), axis=1, keepdims=True)
    v2 = jnp.where(e_iota == i1, neg, masked)
    m2 = jnp.max(v2, axis=1, keepdims=True)
    i2 = jnp.min(jnp.where((v2 == m2) & (e_iota != i1), e_iota, 1e9),
                 axis=1, keepdims=True)

    one1 = e_iota == i1
    one2 = e_iota == i2
    l1 = jnp.sum(jnp.where(one1, lg, 0.0), axis=1, keepdims=True)
    l2 = jnp.sum(jnp.where(one2, lg, 0.0), axis=1, keepdims=True)
    mx = jnp.maximum(l1, l2)
    ex1 = jnp.exp(l1 - mx)
    ex2 = jnp.exp(l2 - mx)
    s = ex1 + ex2
    w1 = ex1 / s
    w2 = ex2 / s

    member = (one1 | one2).astype(jnp.float32)
    rank, counts = colcumsum_excl(member)
    seg = jnp.floor((counts + jnp.float32(TILE - 1)) / TILE) * TILE

    rio8 = lax.broadcasted_iota(jnp.int32, (E, E), 0)
    cio8 = lax.broadcasted_iota(jnp.int32, (E, E), 1)
    slt8 = (rio8 < cio8).astype(jnp.float32)
    offs = jnp.dot(seg, slt8, preferred_element_type=jnp.float32)

    o1 = jnp.sum(jnp.where(one1, offs, 0.0), axis=1, keepdims=True)
    o2 = jnp.sum(jnp.where(one2, offs, 0.0), axis=1, keepdims=True)
    r1 = jnp.sum(jnp.where(one1, rank, 0.0), axis=1, keepdims=True)
    r2 = jnp.sum(jnp.where(one2, rank, 0.0), axis=1, keepdims=True)
    pos1_ref[...] = (o1 + r1).astype(jnp.int32)
    pos2_ref[...] = (o2 + r2).astype(jnp.int32)
    bwr = bwr_ref[...]
    zpad = jnp.zeros((N, 128 - LR - E), jnp.float32)
    cat1_ref[...] = jnp.concatenate(
        [bwr, jnp.broadcast_to(w1, (N, E)), zpad], axis=1)
    cat2_ref[...] = jnp.concatenate(
        [bwr, jnp.broadcast_to(w2, (N, E)), zpad], axis=1)

    ends = offs + seg
    ts = lax.broadcasted_iota(jnp.int32, (64, E), 0).astype(jnp.float32) * TILE
    te = jnp.sum((ts >= ends).astype(jnp.float32), axis=1, keepdims=True)
    te_ref[...] = jnp.minimum(te, jnp.float32(E - 1)).astype(jnp.int32)

    imp = jnp.sum(jnp.where(one1, w1, 0.0) + jnp.where(one2, w2, 0.0),
                  axis=0, keepdims=True)
    load = counts

    def cv(v):
        m = jnp.mean(v, axis=1, keepdims=True)
        var = jnp.mean((v - m) ** 2, axis=1, keepdims=True)
        return var / (m * m + 1e-10)

    loss_ref[...] = (cv(imp) + cv(load)) * LOSS_COEF


def _router(logits, bwr):
    return pl.pallas_call(
        _router_body,
        out_shape=[
            jax.ShapeDtypeStruct((N, 1), jnp.int32),
            jax.ShapeDtypeStruct((N, 1), jnp.int32),
            jax.ShapeDtypeStruct((N, 128), jnp.float32),
            jax.ShapeDtypeStruct((N, 128), jnp.float32),
            jax.ShapeDtypeStruct((64, 1), jnp.int32),
            jax.ShapeDtypeStruct((1, 1), jnp.float32),
        ],
        scratch_shapes=[pltpu.VMEM((N, E), jnp.float32)],
    )(logits, bwr)


@functools.lru_cache(maxsize=1)
def _sc_mesh():
    return plsc.VectorSubcoreMesh(
        core_axis_name="c", subcore_axis_name="s",
        num_cores=SC_CORES, num_subcores=SC_SUBCORES)


@functools.lru_cache(maxsize=1)
def _sc_scatter():
    @functools.partial(
        pl.kernel, mesh=_sc_mesh(),
        out_type=[
            jax.ShapeDtypeStruct((P, C), jnp.float32),
            jax.ShapeDtypeStruct((P, 128), jnp.float32),
        ],
        scratch_types=[
            pltpu.VMEM((RW,), jnp.int32),
            pltpu.VMEM((RW,), jnp.int32),
            pltpu.VMEM((RW, C), jnp.float32),
            pltpu.VMEM((RW, 128), jnp.float32),
            pltpu.VMEM((RW, 128), jnp.float32),
            pltpu.SemaphoreType.DMA,
        ],
    )
    def body(x_hbm, cat1_hbm, cat2_hbm, pos1_hbm, pos2_hbm,
             ys_hbm, meta_hbm, i1_v, i2_v, rows_v, m1_v, m2_v, sem):
        wid = lax.axis_index("s") * SC_CORES + lax.axis_index("c")
        base = wid * RW
        pltpu.sync_copy(x_hbm.at[pl.ds(base, RW)], rows_v)
        pltpu.sync_copy(pos1_hbm.at[pl.ds(base, RW)], i1_v)
        pltpu.sync_copy(pos2_hbm.at[pl.ds(base, RW)], i2_v)
        pltpu.sync_copy(cat1_hbm.at[pl.ds(base, RW)], m1_v)
        pltpu.sync_copy(cat2_hbm.at[pl.ds(base, RW)], m2_v)
        c1 = pltpu.async_copy(rows_v, ys_hbm.at[i1_v], sem)
        c2 = pltpu.async_copy(rows_v, ys_hbm.at[i2_v], sem)
        c3 = pltpu.async_copy(m1_v, meta_hbm.at[i1_v], sem)
        c4 = pltpu.async_copy(m2_v, meta_hbm.at[i2_v], sem)
        c1.wait()
        c2.wait()
        c3.wait()
        c4.wait()

    return body


def _scatter_call(x32, cat1, cat2, pos1f, pos2f):
    return _sc_scatter()(x32, cat1, cat2, pos1f, pos2f)


def _ffn_body(te_ref, ys_ref, meta_ref, W1_ref, b1_ref, A1_ref,
              B1_ref, W2_ref, b2_ref, A2_ref, B2_ref, os_ref):
    ys = ys_ref[...]
    meta = meta_ref[...]
    bwr = meta[:, :LR]
    xa = jnp.dot(ys, A1_ref[0], preferred_element_type=jnp.float32) * bwr
    h = (jnp.dot(ys, W1_ref[0], preferred_element_type=jnp.float32)
         + b1_ref[0]
         + jnp.dot(xa, B1_ref[0], preferred_element_type=jnp.float32))
    h = jax.nn.gelu(h)
    ha = jnp.dot(h, A2_ref[0], preferred_element_type=jnp.float32) * bwr
    out = (jnp.dot(h, W2_ref[0], preferred_element_type=jnp.float32)
           + b2_ref[0]
           + jnp.dot(ha, B2_ref[0], preferred_element_type=jnp.float32))
    os_ref[...] = out * meta[:, LR:LR + 1]


def _ffn(tef, ys, meta, W1, b1r, A1c, B1c, W2, b2r, A2c, B2c):
    grid_spec = pltpu.PrefetchScalarGridSpec(
        num_scalar_prefetch=1,
        grid=(NT,),
        in_specs=[
            pl.BlockSpec((TILE, C), lambda i, te: (i, 0)),
            pl.BlockSpec((TILE, 128), lambda i, te: (i, 0)),
            pl.BlockSpec((1, C, H), lambda i, te: (te[i], 0, 0)),
            pl.BlockSpec((1, 1, H), lambda i, te: (te[i], 0, 0)),
            pl.BlockSpec((1, C, LR), lambda i, te: (te[i], 0, 0)),
            pl.BlockSpec((1, LR, H), lambda i, te: (te[i], 0, 0)),
            pl.BlockSpec((1, H, C), lambda i, te: (te[i], 0, 0)),
            pl.BlockSpec((1, 1, C), lambda i, te: (te[i], 0, 0)),
            pl.BlockSpec((1, H, LR), lambda i, te: (te[i], 0, 0)),
            pl.BlockSpec((1, LR, C), lambda i, te: (te[i], 0, 0)),
        ],
        out_specs=pl.BlockSpec((TILE, C), lambda i, te: (i, 0)),
    )
    return pl.pallas_call(
        _ffn_body,
        grid_spec=grid_spec,
        out_shape=jax.ShapeDtypeStruct((P, C), jnp.float32),
    )(tef, ys, meta, W1, b1r, A1c, B1c, W2, b2r, A2c, B2c)


@functools.lru_cache(maxsize=1)
def _sc_combine():
    @functools.partial(
        pl.kernel, mesh=_sc_mesh(),
        out_type=jax.ShapeDtypeStruct((N, C), jnp.float32),
        scratch_types=[
            pltpu.VMEM((CT,), jnp.int32),
            pltpu.VMEM((CT,), jnp.int32),
            pltpu.VMEM((CT, C), jnp.float32),
            pltpu.VMEM((CT, C), jnp.float32),
            pltpu.VMEM((CT, C), jnp.float32),
            pltpu.SemaphoreType.DMA,
            pltpu.SemaphoreType.DMA,
        ],
    )
    def body(os_hbm, pos1_hbm, pos2_hbm, fin_hbm,
             i1_v, i2_v, b1_v, b2_v, o_v, s1, s2):
        wid = lax.axis_index("s") * SC_CORES + lax.axis_index("c")
        for c in range(RW // CT):
            base = wid * RW + c * CT
            pltpu.sync_copy(pos1_hbm.at[pl.ds(base, CT)], i1_v)
            pltpu.sync_copy(pos2_hbm.at[pl.ds(base, CT)], i2_v)
            cp1 = pltpu.async_copy(os_hbm.at[i1_v], b1_v, s1)
            cp2 = pltpu.async_copy(os_hbm.at[i2_v], b2_v, s2)
            cp1.wait()
            cp2.wait()

            def tok(ti, _):
                for j in range(C // 16):
                    o_v[ti, j * 16:(j + 1) * 16] = (
                        b1_v[ti, j * 16:(j + 1) * 16]
                        + b2_v[ti, j * 16:(j + 1) * 16])
                return 0

            lax.fori_loop(0, CT, tok, 0)
            pltpu.sync_copy(o_v, fin_hbm.at[pl.ds(base, CT)])

    return body


def _combine_call(os, pos1f, pos2f):
    return _sc_combine()(os, pos1f, pos2f)


def kernel(x, band_weights, x_prev_tokens, W_ext, ln_g, ln_b, W_gate, b_gate,
           W1, b1, A1, B1, W2, b2, A2, B2):
    x32 = x.astype(jnp.float32)
    Z = x32 @ W_ext.T
    delta = jax.lax.stop_gradient(x32 - x_prev_tokens.astype(jnp.float32))
    ad = jnp.abs(delta)
    mu = jnp.log1p(jnp.mean(ad, axis=1, keepdims=True))
    sd = jnp.log1p(jnp.std(ad, axis=1, keepdims=True, ddof=1))
    enh = jnp.concatenate([x32, Z, mu, sd], axis=1)
    mean = jnp.mean(enh, axis=-1, keepdims=True)
    var = jnp.var(enh, axis=-1, keepdims=True)
    enh = (enh - mean) / jnp.sqrt(var + 1e-5) * ln_g + ln_b
    logits = enh @ W_gate.T + b_gate

    bwr = jnp.repeat(band_weights, R, axis=1)
    pos1, pos2, cat1, cat2, te, lossv = _router(logits, bwr)
    pos1f = pos1.reshape(N)
    pos2f = pos2.reshape(N)
    tef = te.reshape(64)

    A1c = jnp.transpose(A1, (0, 2, 1, 3)).reshape(E, C, LR)
    B1c = B1.reshape(E, LR, H) * SCALE
    A2c = jnp.transpose(A2, (0, 2, 1, 3)).reshape(E, H, LR)
    B2c = B2.reshape(E, LR, C) * SCALE
    ys, meta = _scatter_call(x32, cat1, cat2, pos1f, pos2f)
    os = _ffn(tef, ys, meta, W1, b1.reshape(E, 1, H), A1c, B1c,
              W2, b2.reshape(E, 1, C), A2c, B2c)
    final = _combine_call(os, pos1f, pos2f)
    return final, lossv.reshape(())
